# baseline (speedup 1.0000x reference)
import jax
import jax.numpy as jnp
from jax import lax
from jax.experimental import pallas as pl
from jax.experimental.pallas import tpu as pltpu
from jax.experimental.pallas import tpu_sc as plsc

_B = 16384
_D = 64
_L = 16
_NC = 2
_NS = 16
_NW = _NC * _NS
_BPW = _B // _NW
_NCHUNK = 4
_CB = _BPW // _NCHUNK
_NG = _CB // _L
_DUNROLL = 8
_GAMMA = 1.0


def _tran_e_body(ph, pt, pr, nh, nt, nr, ent, rel, out,
                 ih, it, ir, jh, jt, jr,
                 ha, ta, ra, hb, tb, rb, accv, sem):
    wid = lax.axis_index("s") * _NC + lax.axis_index("c")
    base = wid * _NCHUNK
    pltpu.sync_copy(ph.at[pl.ds(base, _NCHUNK)], ih)
    pltpu.sync_copy(pt.at[pl.ds(base, _NCHUNK)], it)
    pltpu.sync_copy(pr.at[pl.ds(base, _NCHUNK)], ir)
    pltpu.sync_copy(nh.at[pl.ds(base, _NCHUNK)], jh)
    pltpu.sync_copy(nt.at[pl.ds(base, _NCHUNK)], jt)
    pltpu.sync_copy(nr.at[pl.ds(base, _NCHUNK)], jr)

    iota = lax.iota(jnp.int32, _L)
    wacc = jnp.zeros((_L,), jnp.float32)
    for c in range(_NCHUNK):
        cps = [
            pltpu.async_copy(ent.at[ih.at[c]], ha, sem),
            pltpu.async_copy(ent.at[it.at[c]], ta, sem),
            pltpu.async_copy(rel.at[ir.at[c]], ra, sem),
            pltpu.async_copy(ent.at[jh.at[c]], hb, sem),
            pltpu.async_copy(ent.at[jt.at[c]], tb, sem),
            pltpu.async_copy(rel.at[jr.at[c]], rb, sem),
        ]
        for cp in cps:
            cp.wait()

        def group(g, acc):
            row = g * _L + iota

            def dstep(i, carry):
                pacc, nacc = carry
                d0 = i * _DUNROLL
                for dd in range(_DUNROLL):
                    dv = jnp.full((_L,), d0 + dd, jnp.int32)
                    h1 = plsc.load_gather(ha, [row, dv])
                    t1 = plsc.load_gather(ta, [row, dv])
                    r1 = plsc.load_gather(ra, [row, dv])
                    pacc = pacc + jnp.abs(h1 + r1 - t1)
                    h2 = plsc.load_gather(hb, [row, dv])
                    t2 = plsc.load_gather(tb, [row, dv])
                    r2 = plsc.load_gather(rb, [row, dv])
                    nacc = nacc + jnp.abs(h2 + r2 - t2)
                return pacc, nacc

            zero = jnp.zeros((_L,), jnp.float32)
            pacc, nacc = lax.fori_loop(0, _D // _DUNROLL, dstep, (zero, zero))
            return acc + jnp.maximum(_GAMMA + pacc - nacc, 0.0)

        wacc = lax.fori_loop(0, _NG, group, wacc)

    accv[...] = wacc
    pltpu.sync_copy(accv, out.at[wid])


_sc_tran_e = pl.kernel(
    _tran_e_body,
    out_type=jax.ShapeDtypeStruct((_NW, _L), jnp.float32),
    mesh=plsc.VectorSubcoreMesh(core_axis_name="c", subcore_axis_name="s"),
    compiler_params=pltpu.CompilerParams(
        needs_layout_passes=False, use_tc_tiling_on_sc=False),
    scratch_types=[
        pltpu.VMEM((_NCHUNK, _CB), jnp.int32),
        pltpu.VMEM((_NCHUNK, _CB), jnp.int32),
        pltpu.VMEM((_NCHUNK, _CB), jnp.int32),
        pltpu.VMEM((_NCHUNK, _CB), jnp.int32),
        pltpu.VMEM((_NCHUNK, _CB), jnp.int32),
        pltpu.VMEM((_NCHUNK, _CB), jnp.int32),
        pltpu.VMEM((_CB, _D), jnp.float32),
        pltpu.VMEM((_CB, _D), jnp.float32),
        pltpu.VMEM((_CB, _D), jnp.float32),
        pltpu.VMEM((_CB, _D), jnp.float32),
        pltpu.VMEM((_CB, _D), jnp.float32),
        pltpu.VMEM((_CB, _D), jnp.float32),
        pltpu.VMEM((_L,), jnp.float32),
        pltpu.SemaphoreType.DMA,
    ],
)


def kernel(pos_head, pos_tail, pos_relation, neg_head, neg_tail, neg_relation,
           entity_embedding, relation_embedding):
    shp = (_NW * _NCHUNK, _CB)
    idx = [a.astype(jnp.int32).reshape(shp)
           for a in (pos_head, pos_tail, pos_relation,
                     neg_head, neg_tail, neg_relation)]
    partials = _sc_tran_e(*idx, entity_embedding, relation_embedding)
    return jnp.sum(partials)


# speedup vs baseline: 1.1018x; 1.1018x over previous
import jax
import jax.numpy as jnp
from jax import lax
from jax.experimental import pallas as pl
from jax.experimental.pallas import tpu as pltpu
from jax.experimental.pallas import tpu_sc as plsc

_B = 16384
_D = 64
_L = 16
_NC = 2
_NS = 16
_NW = _NC * _NS
_BPW = _B // _NW
_CB = 64
_NCH = _BPW // _CB
_GAMMA = 1.0


def _tran_e_body(ih, it, ir, jh, jt, jr, oh, ot, orr, qh, qt, qr, entP, relP,
                 out,
                 vh, vt, vr, wh, wt, wr,
                 poh, pot, por, pqh, pqt, pqr,
                 a1, a2, a3, a4, a5, a6,
                 b1, b2, b3, b4, b5, b6,
                 accv, semA, semB):
    wid = lax.axis_index("s") * _NC + lax.axis_index("c")
    pltpu.sync_copy(ih.at[wid], vh)
    pltpu.sync_copy(it.at[wid], vt)
    pltpu.sync_copy(ir.at[wid], vr)
    pltpu.sync_copy(jh.at[wid], wh)
    pltpu.sync_copy(jt.at[wid], wt)
    pltpu.sync_copy(jr.at[wid], wr)
    pltpu.sync_copy(oh.at[wid], poh)
    pltpu.sync_copy(ot.at[wid], pot)
    pltpu.sync_copy(orr.at[wid], por)
    pltpu.sync_copy(qh.at[wid], pqh)
    pltpu.sync_copy(qt.at[wid], pqt)
    pltpu.sync_copy(qr.at[wid], pqr)

    def fire(c, c1, c2, c3, c4, c5, c6, sem):
        pltpu.async_copy(entP.at[vh.at[c]], c1, sem)
        pltpu.async_copy(entP.at[vt.at[c]], c2, sem)
        pltpu.async_copy(relP.at[vr.at[c]], c3, sem)
        pltpu.async_copy(entP.at[wh.at[c]], c4, sem)
        pltpu.async_copy(entP.at[wt.at[c]], c5, sem)
        pltpu.async_copy(relP.at[wr.at[c]], c6, sem)

    def drain(c1, c2, c3, c4, c5, c6, sem):
        src = entP.at[pl.ds(0, _CB)]
        pltpu.make_async_copy(src, c1, sem).wait()
        pltpu.make_async_copy(src, c2, sem).wait()
        pltpu.make_async_copy(src, c3, sem).wait()
        pltpu.make_async_copy(src, c4, sem).wait()
        pltpu.make_async_copy(src, c5, sem).wait()
        pltpu.make_async_copy(src, c6, sem).wait()

    def contrib(c, c1, c2, c3, c4, c5, c6, wsum0):
        def subgroup(sg, wsum):
            sb = c * _CB + sg * _L
            p1 = poh[pl.ds(sb, _L)]
            p2 = pot[pl.ds(sb, _L)]
            p3 = por[pl.ds(sb, _L)]
            p4 = pqh[pl.ds(sb, _L)]
            p5 = pqt[pl.ds(sb, _L)]
            p6 = pqr[pl.ds(sb, _L)]
            row0 = sg * _L
            for k in range(_L):
                row = row0 + k
                pv = jnp.zeros((_L,), jnp.float32)
                nv = jnp.zeros((_L,), jnp.float32)
                for m in range(_D // _L):
                    o = m * _L
                    pv = pv + jnp.abs(c1[row, pl.ds(p1[k] + o, _L)]
                                      + c3[row, pl.ds(p3[k] + o, _L)]
                                      - c2[row, pl.ds(p2[k] + o, _L)])
                    nv = nv + jnp.abs(c4[row, pl.ds(p4[k] + o, _L)]
                                      + c6[row, pl.ds(p6[k] + o, _L)]
                                      - c5[row, pl.ds(p5[k] + o, _L)])
                wsum = wsum + jnp.maximum(
                    _GAMMA + jnp.sum(pv) - jnp.sum(nv), 0.0)
            return wsum

        return lax.fori_loop(0, _CB // _L, subgroup, wsum0)

    fire(0, a1, a2, a3, a4, a5, a6, semA)

    def chunk_pair(i, wsum):
        ca = 2 * i
        fire(ca + 1, b1, b2, b3, b4, b5, b6, semB)
        drain(a1, a2, a3, a4, a5, a6, semA)
        wsum = contrib(ca, a1, a2, a3, a4, a5, a6, wsum)
        fire(ca + 2, a1, a2, a3, a4, a5, a6, semA)
        drain(b1, b2, b3, b4, b5, b6, semB)
        return contrib(ca + 1, b1, b2, b3, b4, b5, b6, wsum)

    wsum = lax.fori_loop(0, _NCH // 2 - 1, chunk_pair, jnp.float32(0.0))

    fire(_NCH - 1, b1, b2, b3, b4, b5, b6, semB)
    drain(a1, a2, a3, a4, a5, a6, semA)
    wsum = contrib(_NCH - 2, a1, a2, a3, a4, a5, a6, wsum)
    drain(b1, b2, b3, b4, b5, b6, semB)
    wsum = contrib(_NCH - 1, b1, b2, b3, b4, b5, b6, wsum)

    lane = lax.iota(jnp.int32, _L)
    accv[...] = jnp.where(lane == 0, wsum, 0.0)
    pltpu.sync_copy(accv, out.at[wid])


_idx32 = [pltpu.VMEM((_NCH, _CB), jnp.int32)] * 6
_poff = [pltpu.VMEM((_BPW,), jnp.int32)] * 6
_rowbuf = [pltpu.VMEM((_CB, 2 * _D), jnp.float32)] * 12

_sc_tran_e = pl.kernel(
    _tran_e_body,
    out_type=jax.ShapeDtypeStruct((_NW, _L), jnp.float32),
    mesh=plsc.VectorSubcoreMesh(core_axis_name="c", subcore_axis_name="s"),
    compiler_params=pltpu.CompilerParams(needs_layout_passes=False),
    scratch_types=[*_idx32, *_poff, *_rowbuf,
                   pltpu.VMEM((_L,), jnp.float32),
                   pltpu.SemaphoreType.DMA,
                   pltpu.SemaphoreType.DMA],
)


def kernel(pos_head, pos_tail, pos_relation, neg_head, neg_tail, neg_relation,
           entity_embedding, relation_embedding):
    srcs = (pos_head, pos_tail, pos_relation,
            neg_head, neg_tail, neg_relation)
    half = [jnp.right_shift(a, 1).astype(jnp.int32).reshape(_NW, _NCH, _CB)
            for a in srcs]
    poff = [(jnp.bitwise_and(a, 1) * _D).astype(jnp.int32).reshape(_NW, _BPW)
            for a in srcs]
    entP = entity_embedding.reshape(500000, 2 * _D)
    relP = relation_embedding.reshape(500, 2 * _D)
    partials = _sc_tran_e(*half, *poff, entP, relP)
    return jnp.sum(partials)


# speedup vs baseline: 1.2490x; 1.1337x over previous
import jax
import jax.numpy as jnp
from jax import lax
from jax.experimental import pallas as pl
from jax.experimental.pallas import tpu as pltpu
from jax.experimental.pallas import tpu_sc as plsc

_B = 16384
_D = 64
_L = 16
_NC = 2
_NS = 16
_NW = _NC * _NS
_BPW = _B // _NW
_CB = 64
_NCH = _BPW // _CB
_GAMMA = 1.0


def _tran_e_body(ih, it, ir, jh, jt, jr, entP, relP, out,
                 vh, vt, vr, wh, wt, wr,
                 a1, a2, a3, a4, a5, a6,
                 b1, b2, b3, b4, b5, b6,
                 accv, semA, semB):
    wid = lax.axis_index("s") * _NC + lax.axis_index("c")
    pltpu.sync_copy(ih.at[wid], vh)
    pltpu.sync_copy(it.at[wid], vt)
    pltpu.sync_copy(ir.at[wid], vr)
    pltpu.sync_copy(jh.at[wid], wh)
    pltpu.sync_copy(jt.at[wid], wt)
    pltpu.sync_copy(jr.at[wid], wr)

    def fire(c, c1, c2, c3, c4, c5, c6, sem):
        pltpu.async_copy(entP.at[vh.at[c]], c1, sem)
        pltpu.async_copy(entP.at[vt.at[c]], c2, sem)
        pltpu.async_copy(relP.at[vr.at[c]], c3, sem)
        pltpu.async_copy(entP.at[wh.at[c]], c4, sem)
        pltpu.async_copy(entP.at[wt.at[c]], c5, sem)
        pltpu.async_copy(relP.at[wr.at[c]], c6, sem)

    def drain(c1, c2, c3, c4, c5, c6, sem):
        src = entP.at[pl.ds(0, _CB)]
        pltpu.make_async_copy(src, c1, sem).wait()
        pltpu.make_async_copy(src, c2, sem).wait()
        pltpu.make_async_copy(src, c3, sem).wait()
        pltpu.make_async_copy(src, c4, sem).wait()
        pltpu.make_async_copy(src, c5, sem).wait()
        pltpu.make_async_copy(src, c6, sem).wait()

    def contrib(c, c1, c2, c3, c4, c5, c6, wsum0):
        def subgroup(sg, wsum):
            row0 = sg * _L
            for k in range(_L):
                row = row0 + k
                pv = jnp.zeros((_L,), jnp.float32)
                nv = jnp.zeros((_L,), jnp.float32)
                for m in range(_D // _L):
                    o = m * _L
                    sl = pl.ds(o, _L)
                    pv = pv + jnp.abs(c1[row, sl] + c3[row, sl] - c2[row, sl])
                    nv = nv + jnp.abs(c4[row, sl] + c6[row, sl] - c5[row, sl])
                wsum = wsum + jnp.maximum(
                    _GAMMA + jnp.sum(pv) - jnp.sum(nv), 0.0)
            return wsum

        return lax.fori_loop(0, _CB // _L, subgroup, wsum0)

    fire(0, a1, a2, a3, a4, a5, a6, semA)

    def chunk_pair(i, wsum):
        ca = 2 * i
        fire(ca + 1, b1, b2, b3, b4, b5, b6, semB)
        drain(a1, a2, a3, a4, a5, a6, semA)
        wsum = contrib(ca, a1, a2, a3, a4, a5, a6, wsum)
        fire(ca + 2, a1, a2, a3, a4, a5, a6, semA)
        drain(b1, b2, b3, b4, b5, b6, semB)
        return contrib(ca + 1, b1, b2, b3, b4, b5, b6, wsum)

    wsum = lax.fori_loop(0, _NCH // 2 - 1, chunk_pair, jnp.float32(0.0))

    fire(_NCH - 1, b1, b2, b3, b4, b5, b6, semB)
    drain(a1, a2, a3, a4, a5, a6, semA)
    wsum = contrib(_NCH - 2, a1, a2, a3, a4, a5, a6, wsum)
    drain(b1, b2, b3, b4, b5, b6, semB)
    wsum = contrib(_NCH - 1, b1, b2, b3, b4, b5, b6, wsum)

    lane = lax.iota(jnp.int32, _L)
    accv[...] = jnp.where(lane == 0, wsum, 0.0)
    pltpu.sync_copy(accv, out.at[wid])


_idx32 = [pltpu.VMEM((_NCH, _CB), jnp.int32)] * 6
_rowbuf = [pltpu.VMEM((_CB, 2 * _D), jnp.float32)] * 12

_sc_tran_e = pl.kernel(
    _tran_e_body,
    out_type=jax.ShapeDtypeStruct((_NW, _L), jnp.float32),
    mesh=plsc.VectorSubcoreMesh(core_axis_name="c", subcore_axis_name="s"),
    compiler_params=pltpu.CompilerParams(needs_layout_passes=False),
    scratch_types=[*_idx32, *_rowbuf,
                   pltpu.VMEM((_L,), jnp.float32),
                   pltpu.SemaphoreType.DMA,
                   pltpu.SemaphoreType.DMA],
)


def kernel(pos_head, pos_tail, pos_relation, neg_head, neg_tail, neg_relation,
           entity_embedding, relation_embedding):
    srcs = (pos_head, pos_tail, pos_relation,
            neg_head, neg_tail, neg_relation)
    idx = [a.astype(jnp.int32).reshape(_NW, _NCH, _CB) for a in srcs]
    entP = jnp.pad(entity_embedding, ((0, 0), (0, _D)))
    relP = jnp.pad(relation_embedding, ((0, 0), (0, _D)))
    partials = _sc_tran_e(*idx, entP, relP)
    return jnp.sum(partials)


# speedup vs baseline: 2.2225x; 1.7793x over previous
import jax
import jax.numpy as jnp
from jax import lax
from jax.experimental import pallas as pl
from jax.experimental.pallas import tpu as pltpu
from jax.experimental.pallas import tpu_sc as plsc

_B = 16384
_D = 64
_L = 16
_NC = 2
_NS = 16
_NW = _NC * _NS
_BPW = _B // _NW
_CB = 64
_NCH = _BPW // _CB
_GAMMA = 1.0


def _tran_e_body(ih, it, ir, jh, jt, jr, entP, relP, out,
                 vh, vt, vr, wh, wt, wr,
                 a1, a2, a3, a4, a5, a6,
                 b1, b2, b3, b4, b5, b6,
                 accv, semA, semB):
    wid = lax.axis_index("s") * _NC + lax.axis_index("c")
    pltpu.sync_copy(ih.at[wid], vh)
    pltpu.sync_copy(it.at[wid], vt)
    pltpu.sync_copy(ir.at[wid], vr)
    pltpu.sync_copy(jh.at[wid], wh)
    pltpu.sync_copy(jt.at[wid], wt)
    pltpu.sync_copy(jr.at[wid], wr)

    def fire(c, c1, c2, c3, c4, c5, c6, sem):
        pltpu.async_copy(entP.at[vh.at[c]], c1, sem)
        pltpu.async_copy(entP.at[vt.at[c]], c2, sem)
        pltpu.async_copy(relP.at[vr.at[c]], c3, sem)
        pltpu.async_copy(entP.at[wh.at[c]], c4, sem)
        pltpu.async_copy(entP.at[wt.at[c]], c5, sem)
        pltpu.async_copy(relP.at[wr.at[c]], c6, sem)

    def drain(c1, c2, c3, c4, c5, c6, sem):
        src = entP.at[pl.ds(0, _CB)]
        pltpu.make_async_copy(src, c1, sem).wait()
        pltpu.make_async_copy(src, c2, sem).wait()
        pltpu.make_async_copy(src, c3, sem).wait()
        pltpu.make_async_copy(src, c4, sem).wait()
        pltpu.make_async_copy(src, c5, sem).wait()
        pltpu.make_async_copy(src, c6, sem).wait()

    def contrib(c, c1, c2, c3, c4, c5, c6, wsum0):
        def subgroup(sg, wsum):
            row0 = sg * _L
            for k in range(_L):
                row = row0 + k
                pv = jnp.zeros((_L,), jnp.float32)
                nv = jnp.zeros((_L,), jnp.float32)
                for m in range(_D // _L):
                    o = m * _L
                    sl = pl.ds(o, _L)
                    pv = pv + jnp.abs(c1[row, sl] + c3[row, sl] - c2[row, sl])
                    nv = nv + jnp.abs(c4[row, sl] + c6[row, sl] - c5[row, sl])
                wsum = wsum + jnp.maximum(
                    _GAMMA + jnp.sum(pv) - jnp.sum(nv), 0.0)
            return wsum

        return lax.fori_loop(0, _CB // _L, subgroup, wsum0)

    fire(0, a1, a2, a3, a4, a5, a6, semA)

    def chunk_pair(i, wsum):
        ca = 2 * i
        fire(ca + 1, b1, b2, b3, b4, b5, b6, semB)
        drain(a1, a2, a3, a4, a5, a6, semA)
        wsum = contrib(ca, a1, a2, a3, a4, a5, a6, wsum)
        fire(ca + 2, a1, a2, a3, a4, a5, a6, semA)
        drain(b1, b2, b3, b4, b5, b6, semB)
        return contrib(ca + 1, b1, b2, b3, b4, b5, b6, wsum)

    wsum = lax.fori_loop(0, _NCH // 2 - 1, chunk_pair, jnp.float32(0.0))

    fire(_NCH - 1, b1, b2, b3, b4, b5, b6, semB)
    drain(a1, a2, a3, a4, a5, a6, semA)
    wsum = contrib(_NCH - 2, a1, a2, a3, a4, a5, a6, wsum)
    drain(b1, b2, b3, b4, b5, b6, semB)
    wsum = contrib(_NCH - 1, b1, b2, b3, b4, b5, b6, wsum)

    lane = lax.iota(jnp.int32, _L)
    accv[...] = jnp.where(lane == 0, wsum, 0.0)
    pltpu.sync_copy(accv, out.at[wid])


_idx32 = [pltpu.VMEM((_NCH, _CB), jnp.int32)] * 6
_rowbuf = [pltpu.VMEM((_CB, 2 * _D), jnp.float32)] * 12

_sc_tran_e = pl.kernel(
    _tran_e_body,
    out_type=jax.ShapeDtypeStruct((_NW, _L), jnp.float32),
    mesh=plsc.VectorSubcoreMesh(core_axis_name="c", subcore_axis_name="s"),
    compiler_params=pltpu.CompilerParams(needs_layout_passes=False),
    scratch_types=[*_idx32, *_rowbuf,
                   pltpu.VMEM((_L,), jnp.float32),
                   pltpu.SemaphoreType.DMA,
                   pltpu.SemaphoreType.DMA],
)


_E = 1000000
_CTC = 8192
_TCG = (_E + _CTC - 1) // _CTC


def _pack_body(src_ref, dst_ref):
    dst_ref[:, pl.ds(0, _D)] = src_ref[...].T


_tc_pack = pl.pallas_call(
    _pack_body,
    grid=(_TCG,),
    in_specs=[pl.BlockSpec((_D, _CTC), lambda i: (0, i))],
    out_specs=pl.BlockSpec((_CTC, 2 * _D), lambda i: (i, 0)),
    out_shape=jax.ShapeDtypeStruct((_E, 2 * _D), jnp.float32),
)


def kernel(pos_head, pos_tail, pos_relation, neg_head, neg_tail, neg_relation,
           entity_embedding, relation_embedding):
    srcs = (pos_head, pos_tail, pos_relation,
            neg_head, neg_tail, neg_relation)
    idx = [a.astype(jnp.int32).reshape(_NW, _NCH, _CB) for a in srcs]
    entP = _tc_pack(entity_embedding.T)
    relP = jnp.pad(relation_embedding, ((0, 0), (0, _D)))
    partials = _sc_tran_e(*idx, entP, relP)
    return jnp.sum(partials)


# speedup vs baseline: 2.3659x; 1.0645x over previous
import jax
import jax.numpy as jnp
from jax import lax
from jax.experimental import pallas as pl
from jax.experimental.pallas import tpu as pltpu
from jax.experimental.pallas import tpu_sc as plsc

_B = 16384
_D = 64
_L = 16
_NC = 2
_NS = 16
_NW = _NC * _NS
_BPW = _B // _NW
_CB = 64
_NCH = _BPW // _CB
_GAMMA = 1.0


def _tran_e_body(ih, it, ir, jh, jt, jr, entP, relP, out,
                 vh, vt, vr, wh, wt, wr,
                 a1, a2, a3, a4, a5, a6,
                 b1, b2, b3, b4, b5, b6,
                 accv, semA, semB):
    wid = lax.axis_index("s") * _NC + lax.axis_index("c")
    pltpu.sync_copy(ih.at[wid], vh)
    pltpu.sync_copy(it.at[wid], vt)
    pltpu.sync_copy(ir.at[wid], vr)
    pltpu.sync_copy(jh.at[wid], wh)
    pltpu.sync_copy(jt.at[wid], wt)
    pltpu.sync_copy(jr.at[wid], wr)

    def fire(c, c1, c2, c3, c4, c5, c6, sem):
        pltpu.async_copy(entP.at[vh.at[c]], c1, sem)
        pltpu.async_copy(entP.at[vt.at[c]], c2, sem)
        pltpu.async_copy(relP.at[vr.at[c]], c3, sem)
        pltpu.async_copy(entP.at[wh.at[c]], c4, sem)
        pltpu.async_copy(entP.at[wt.at[c]], c5, sem)
        pltpu.async_copy(relP.at[wr.at[c]], c6, sem)

    def drain(c1, c2, c3, c4, c5, c6, sem):
        src = entP.at[pl.ds(0, _CB)]
        pltpu.make_async_copy(src, c1, sem).wait()
        pltpu.make_async_copy(src, c2, sem).wait()
        pltpu.make_async_copy(src, c3, sem).wait()
        pltpu.make_async_copy(src, c4, sem).wait()
        pltpu.make_async_copy(src, c5, sem).wait()
        pltpu.make_async_copy(src, c6, sem).wait()

    def contrib(c, c1, c2, c3, c4, c5, c6, wsum0):
        def subgroup(sg, wsum):
            row0 = sg * _L
            for k in range(_L):
                row = row0 + k
                pv = jnp.zeros((_L,), jnp.float32)
                nv = jnp.zeros((_L,), jnp.float32)
                for m in range(_D // _L):
                    o = m * _L
                    sl = pl.ds(o, _L)
                    pv = pv + jnp.abs(c1[row, sl] + c3[row, sl] - c2[row, sl])
                    nv = nv + jnp.abs(c4[row, sl] + c6[row, sl] - c5[row, sl])
                wsum = wsum + jnp.maximum(
                    _GAMMA + jnp.sum(pv) - jnp.sum(nv), 0.0)
            return wsum

        return lax.fori_loop(0, _CB // _L, subgroup, wsum0)

    fire(0, a1, a2, a3, a4, a5, a6, semA)

    def chunk_pair(i, wsum):
        ca = 2 * i
        fire(ca + 1, b1, b2, b3, b4, b5, b6, semB)
        drain(a1, a2, a3, a4, a5, a6, semA)
        wsum = contrib(ca, a1, a2, a3, a4, a5, a6, wsum)
        fire(ca + 2, a1, a2, a3, a4, a5, a6, semA)
        drain(b1, b2, b3, b4, b5, b6, semB)
        return contrib(ca + 1, b1, b2, b3, b4, b5, b6, wsum)

    wsum = lax.fori_loop(0, _NCH // 2 - 1, chunk_pair, jnp.float32(0.0))

    fire(_NCH - 1, b1, b2, b3, b4, b5, b6, semB)
    drain(a1, a2, a3, a4, a5, a6, semA)
    wsum = contrib(_NCH - 2, a1, a2, a3, a4, a5, a6, wsum)
    drain(b1, b2, b3, b4, b5, b6, semB)
    wsum = contrib(_NCH - 1, b1, b2, b3, b4, b5, b6, wsum)

    lane = lax.iota(jnp.int32, _L)
    accv[...] = jnp.where(lane == 0, wsum, 0.0)
    pltpu.sync_copy(accv, out.at[wid])


_idx32 = [pltpu.VMEM((_NCH, _CB), jnp.int32)] * 6
_rowbuf = [pltpu.VMEM((_CB, 2 * _D), jnp.float32)] * 12

_sc_tran_e = pl.kernel(
    _tran_e_body,
    out_type=jax.ShapeDtypeStruct((_NW, _L), jnp.float32),
    mesh=plsc.VectorSubcoreMesh(core_axis_name="c", subcore_axis_name="s"),
    compiler_params=pltpu.CompilerParams(needs_layout_passes=False),
    scratch_types=[*_idx32, *_rowbuf,
                   pltpu.VMEM((_L,), jnp.float32),
                   pltpu.SemaphoreType.DMA,
                   pltpu.SemaphoreType.DMA],
)


_E = 1000000
_CTC = 16384
_TCG = (_E + _CTC - 1) // _CTC


def _pack_body(src_ref, dst_ref):
    dst_ref[:, pl.ds(0, _D)] = src_ref[...].T


_tc_pack = pl.pallas_call(
    _pack_body,
    grid=(_TCG,),
    in_specs=[pl.BlockSpec((_D, _CTC), lambda i: (0, i))],
    out_specs=pl.BlockSpec((_CTC, 2 * _D), lambda i: (i, 0)),
    out_shape=jax.ShapeDtypeStruct((_E, 2 * _D), jnp.float32),
)


def kernel(pos_head, pos_tail, pos_relation, neg_head, neg_tail, neg_relation,
           entity_embedding, relation_embedding):
    srcs = (pos_head, pos_tail, pos_relation,
            neg_head, neg_tail, neg_relation)
    idx = [a.astype(jnp.int32).reshape(_NW, _NCH, _CB) for a in srcs]
    entP = _tc_pack(entity_embedding.T)
    relP = jnp.pad(relation_embedding, ((0, 0), (0, _D)))
    partials = _sc_tran_e(*idx, entP, relP)
    return jnp.sum(partials)


# speedup vs baseline: 2.4193x; 1.0226x over previous
import jax
import jax.numpy as jnp
from jax import lax
from jax.experimental import pallas as pl
from jax.experimental.pallas import tpu as pltpu
from jax.experimental.pallas import tpu_sc as plsc

_B = 16384
_D = 64
_L = 16
_NC = 2
_NS = 16
_NW = _NC * _NS
_BPW = _B // _NW
_CB = 64
_NCH = _BPW // _CB
_GAMMA = 1.0


def _tran_e_body(ih, it, ir, jh, jt, jr, entP, relP, out,
                 vh, vt, vr, wh, wt, wr,
                 a1, a2, a3, a4, a5, a6,
                 b1, b2, b3, b4, b5, b6,
                 accv, semA, semB):
    wid = lax.axis_index("s") * _NC + lax.axis_index("c")
    pltpu.sync_copy(ih.at[wid], vh)
    pltpu.sync_copy(it.at[wid], vt)
    pltpu.sync_copy(ir.at[wid], vr)
    pltpu.sync_copy(jh.at[wid], wh)
    pltpu.sync_copy(jt.at[wid], wt)
    pltpu.sync_copy(jr.at[wid], wr)

    def fire(c, c1, c2, c3, c4, c5, c6, sem):
        pltpu.async_copy(entP.at[vh.at[c]], c1, sem)
        pltpu.async_copy(entP.at[vt.at[c]], c2, sem)
        pltpu.async_copy(relP.at[vr.at[c]], c3, sem)
        pltpu.async_copy(entP.at[wh.at[c]], c4, sem)
        pltpu.async_copy(entP.at[wt.at[c]], c5, sem)
        pltpu.async_copy(relP.at[wr.at[c]], c6, sem)

    def drain(c1, c2, c3, c4, c5, c6, sem):
        src = entP.at[pl.ds(0, _CB)]
        pltpu.make_async_copy(src, c1, sem).wait()
        pltpu.make_async_copy(src, c2, sem).wait()
        pltpu.make_async_copy(src, c3, sem).wait()
        pltpu.make_async_copy(src, c4, sem).wait()
        pltpu.make_async_copy(src, c5, sem).wait()
        pltpu.make_async_copy(src, c6, sem).wait()

    def contrib(c, c1, c2, c3, c4, c5, c6, wsum0):
        def subgroup(sg, wsum):
            row0 = sg * _L
            for k in range(_L):
                row = row0 + k
                pv = jnp.zeros((_L,), jnp.float32)
                nv = jnp.zeros((_L,), jnp.float32)
                for m in range(_D // _L):
                    o = m * _L
                    sl = pl.ds(o, _L)
                    pv = pv + jnp.abs(c1[row, sl] + c3[row, sl] - c2[row, sl])
                    nv = nv + jnp.abs(c4[row, sl] + c6[row, sl] - c5[row, sl])
                wsum = wsum + jnp.maximum(
                    _GAMMA + jnp.sum(pv) - jnp.sum(nv), 0.0)
            return wsum

        return lax.fori_loop(0, _CB // _L, subgroup, wsum0)

    fire(0, a1, a2, a3, a4, a5, a6, semA)

    def chunk_pair(i, wsum):
        ca = 2 * i
        fire(ca + 1, b1, b2, b3, b4, b5, b6, semB)
        drain(a1, a2, a3, a4, a5, a6, semA)
        wsum = contrib(ca, a1, a2, a3, a4, a5, a6, wsum)
        fire(ca + 2, a1, a2, a3, a4, a5, a6, semA)
        drain(b1, b2, b3, b4, b5, b6, semB)
        return contrib(ca + 1, b1, b2, b3, b4, b5, b6, wsum)

    wsum = lax.fori_loop(0, _NCH // 2 - 1, chunk_pair, jnp.float32(0.0))

    fire(_NCH - 1, b1, b2, b3, b4, b5, b6, semB)
    drain(a1, a2, a3, a4, a5, a6, semA)
    wsum = contrib(_NCH - 2, a1, a2, a3, a4, a5, a6, wsum)
    drain(b1, b2, b3, b4, b5, b6, semB)
    wsum = contrib(_NCH - 1, b1, b2, b3, b4, b5, b6, wsum)

    lane = lax.iota(jnp.int32, _L)
    accv[...] = jnp.where(lane == 0, wsum, 0.0)
    pltpu.sync_copy(accv, out.at[wid])


_idx32 = [pltpu.VMEM((_NCH, _CB), jnp.int32)] * 6
_rowbuf = [pltpu.VMEM((_CB, 2 * _D), jnp.float32)] * 12

_sc_tran_e = pl.kernel(
    _tran_e_body,
    out_type=jax.ShapeDtypeStruct((_NW, _L), jnp.float32),
    mesh=plsc.VectorSubcoreMesh(core_axis_name="c", subcore_axis_name="s"),
    compiler_params=pltpu.CompilerParams(needs_layout_passes=False),
    scratch_types=[*_idx32, *_rowbuf,
                   pltpu.VMEM((_L,), jnp.float32),
                   pltpu.SemaphoreType.DMA,
                   pltpu.SemaphoreType.DMA],
)


_E = 1000000
_CTC = 32768
_TCG = (_E + _CTC - 1) // _CTC


def _pack_body(src_ref, dst_ref):
    dst_ref[:, pl.ds(0, _D)] = src_ref[...].T


_tc_pack = pl.pallas_call(
    _pack_body,
    grid=(_TCG,),
    in_specs=[pl.BlockSpec((_D, _CTC), lambda i: (0, i))],
    out_specs=pl.BlockSpec((_CTC, 2 * _D), lambda i: (i, 0)),
    out_shape=jax.ShapeDtypeStruct((_E, 2 * _D), jnp.float32),
)


def kernel(pos_head, pos_tail, pos_relation, neg_head, neg_tail, neg_relation,
           entity_embedding, relation_embedding):
    srcs = (pos_head, pos_tail, pos_relation,
            neg_head, neg_tail, neg_relation)
    idx = [a.astype(jnp.int32).reshape(_NW, _NCH, _CB) for a in srcs]
    entP = _tc_pack(entity_embedding.T)
    relP = jnp.pad(relation_embedding, ((0, 0), (0, _D)))
    partials = _sc_tran_e(*idx, entP, relP)
    return jnp.sum(partials)
